# Initial kernel scaffold; baseline (speedup 1.0000x reference)
import jax
import jax.numpy as jnp
from jax.experimental import pallas as pl


def kernel(sequencee, table):
    raise NotImplementedError("write your pallas kernel here")


# speedup vs baseline: 6.3473x; 6.3473x over previous
import functools

import jax
import jax.numpy as jnp
from jax import lax
from jax.experimental import pallas as pl
from jax.experimental.pallas import tpu as pltpu
from jax.experimental.pallas import tpu_sc as plsc

NUM_CORES = 2
NUM_SUBCORES = 16
NW = NUM_CORES * NUM_SUBCORES
CHUNK = 128


@functools.lru_cache(maxsize=None)
def _build_gather(n_rows: int, d_model: int):
    assert n_rows % (NW * CHUNK) == 0
    b_per_w = n_rows // NW
    n_steps = b_per_w // CHUNK
    mesh = plsc.VectorSubcoreMesh(
        core_axis_name="c", subcore_axis_name="s", num_cores=NUM_CORES
    )

    @functools.partial(
        pl.kernel,
        out_type=jax.ShapeDtypeStruct((n_rows, d_model), jnp.float32),
        mesh=mesh,
        scratch_types=[
            pltpu.VMEM((b_per_w,), jnp.int32),
            pltpu.VMEM((CHUNK, d_model), jnp.float32),
            pltpu.SemaphoreType.DMA,
        ],
    )
    def gather_kernel(idx_hbm, table_hbm, out_hbm, idx_v, rows_v, sem):
        wid = lax.axis_index("s") * NUM_CORES + lax.axis_index("c")
        base = wid * b_per_w
        pltpu.sync_copy(idx_hbm.at[pl.ds(base, b_per_w)], idx_v)

        @pl.loop(0, n_steps)
        def _step(j):
            off = j * CHUNK
            pltpu.async_copy(
                table_hbm.at[idx_v.at[pl.ds(off, CHUNK)]], rows_v, sem
            ).wait()
            pltpu.sync_copy(rows_v, out_hbm.at[pl.ds(base + off, CHUNK)])

    return gather_kernel


def kernel(sequencee, table):
    b, s = sequencee.shape
    v, d = table.shape
    flat_idx = sequencee.reshape(b * s).astype(jnp.int32)
    out = _build_gather(b * s, d)(flat_idx, table)
    return out.reshape(b, s, d)


# speedup vs baseline: 9.2290x; 1.4540x over previous
import functools

import jax
import jax.numpy as jnp
from jax import lax
from jax.experimental import pallas as pl
from jax.experimental.pallas import tpu as pltpu
from jax.experimental.pallas import tpu_sc as plsc

NUM_CORES = 2
NUM_SUBCORES = 16
NW = NUM_CORES * NUM_SUBCORES
CHUNK = 128


@functools.lru_cache(maxsize=None)
def _build_gather(n_rows: int, d_model: int):
    assert n_rows % (NW * CHUNK) == 0
    b_per_w = n_rows // NW
    n_steps = b_per_w // CHUNK
    mesh = plsc.VectorSubcoreMesh(
        core_axis_name="c", subcore_axis_name="s", num_cores=NUM_CORES
    )

    @functools.partial(
        pl.kernel,
        out_type=jax.ShapeDtypeStruct((n_rows, d_model), jnp.float32),
        mesh=mesh,
        scratch_types=[
            pltpu.VMEM((b_per_w,), jnp.int32),
            pltpu.VMEM((CHUNK, d_model), jnp.float32),
            pltpu.VMEM((CHUNK, d_model), jnp.float32),
            pltpu.SemaphoreType.DMA,
            pltpu.SemaphoreType.DMA,
        ],
    )
    def gather_kernel(idx_hbm, table_hbm, out_hbm, idx_v, rows_a, rows_b, sem_a, sem_b):
        wid = lax.axis_index("s") * NUM_CORES + lax.axis_index("c")
        base = wid * b_per_w
        pltpu.sync_copy(idx_hbm.at[pl.ds(base, b_per_w)], idx_v)

        def gather_src(j):
            return table_hbm.at[idx_v.at[pl.ds(j * CHUNK, CHUNK)]]

        pltpu.async_copy(gather_src(0), rows_a, sem_a)
        pltpu.async_copy(gather_src(1), rows_b, sem_b)

        @pl.loop(0, n_steps // 2)
        def _pair(p):
            j = 2 * p
            pltpu.make_async_copy(gather_src(j), rows_a, sem_a).wait()
            pltpu.sync_copy(rows_a, out_hbm.at[pl.ds(base + j * CHUNK, CHUNK)])

            @pl.when(j + 2 < n_steps)
            def _():
                pltpu.async_copy(gather_src(j + 2), rows_a, sem_a)

            pltpu.make_async_copy(gather_src(j + 1), rows_b, sem_b).wait()
            pltpu.sync_copy(rows_b, out_hbm.at[pl.ds(base + (j + 1) * CHUNK, CHUNK)])

            @pl.when(j + 3 < n_steps)
            def _():
                pltpu.async_copy(gather_src(j + 3), rows_b, sem_b)

    return gather_kernel


def kernel(sequencee, table):
    b, s = sequencee.shape
    v, d = table.shape
    flat_idx = sequencee.reshape(b * s).astype(jnp.int32)
    out = _build_gather(b * s, d)(flat_idx, table)
    return out.reshape(b, s, d)
